# Initial kernel scaffold; baseline (speedup 1.0000x reference)
import functools

import jax
import jax.numpy as jnp
from jax.experimental import pallas as pl
from jax.experimental.pallas import tpu as pltpu

HIDDEN = 768
INTER = 768
NUM_EXPERTS = 8
ALPHA = 1.702
LIMIT = 7.0
TOKENS = 2048
CHUNK = 512


def _router_body(x_ref, rw_ref, rb_ref, scores_ref):
    x = x_ref[...]
    rw = rw_ref[...]
    logits = jax.lax.dot_general(
        x, rw, (((1,), (1,)), ((), ())), preferred_element_type=jnp.float32
    ) + rb_ref[...]
    T, E = logits.shape
    eids = jax.lax.broadcasted_iota(jnp.int32, (T, E), 1)
    m1 = jnp.max(logits, axis=1, keepdims=True)
    i1 = jnp.min(jnp.where(logits == m1, eids, E), axis=1, keepdims=True)
    masked = jnp.where(eids == i1, -jnp.inf, logits)
    m2 = jnp.max(masked, axis=1, keepdims=True)
    i2 = jnp.min(jnp.where(masked == m2, eids, E), axis=1, keepdims=True)
    e2 = jnp.exp(m2 - m1)
    denom = 1.0 + e2
    p1 = 1.0 / denom
    p2 = e2 / denom
    scores_ref[...] = jnp.where(eids == i1, p1, 0.0) + jnp.where(eids == i2, p2, 0.0)


def _expert_body(scores_ref, x_ref, wg_ref, bg_ref, wu_ref, bu_ref,
                 wd_ref, bd_ref, out_ref):
    e = pl.program_id(0)
    wg = wg_ref[0]
    wu = wu_ref[0]
    wd = wd_ref[0]
    bg = bg_ref[0]
    bu = bu_ref[0]
    bd = bd_ref[0]
    T, E = scores_ref.shape
    eids = jax.lax.broadcasted_iota(jnp.int32, (T, E), 1)
    col_full = jnp.sum(
        jnp.where(eids == e, scores_ref[...], 0.0), axis=1, keepdims=True
    )
    for c in range(TOKENS // CHUNK):
        sl = pl.ds(c * CHUNK, CHUNK)
        xc = x_ref[sl, :]
        g = jax.lax.dot_general(
            xc, wg, (((1,), (0,)), ((), ())), preferred_element_type=jnp.float32
        ) + bg
        u = jax.lax.dot_general(
            xc, wu, (((1,), (0,)), ((), ())), preferred_element_type=jnp.float32
        ) + bu
        g = jnp.minimum(g, LIMIT)
        u = jnp.clip(u, -LIMIT, LIMIT)
        glu = g * jax.nn.sigmoid(g * ALPHA)
        h = (u + 1.0) * glu
        y = jax.lax.dot_general(
            h, wd, (((1,), (0,)), ((), ())), preferred_element_type=jnp.float32
        ) + bd
        y = y * col_full[sl, :]

        @pl.when(e == 0)
        def _():
            out_ref[sl, :] = y

        @pl.when(e > 0)
        def _():
            out_ref[sl, :] = out_ref[sl, :] + y


def kernel(hidden_states, router_weight, router_bias, gate_up_proj,
           gate_up_proj_bias, down_proj, down_proj_bias):
    B, S, H = hidden_states.shape
    T = B * S
    E = NUM_EXPERTS
    F = INTER
    hs = hidden_states.reshape(T, H)

    scores = pl.pallas_call(
        _router_body,
        out_shape=jax.ShapeDtypeStruct((T, E), jnp.float32),
    )(hs, router_weight, router_bias.reshape(1, E))

    wg = gate_up_proj[:, :, 0::2]
    wu = gate_up_proj[:, :, 1::2]
    bg = gate_up_proj_bias[:, 0::2].reshape(E, 1, F)
    bu = gate_up_proj_bias[:, 1::2].reshape(E, 1, F)
    bd = down_proj_bias.reshape(E, 1, H)

    out = pl.pallas_call(
        _expert_body,
        grid=(E,),
        in_specs=[
            pl.BlockSpec((T, E), lambda e: (0, 0)),
            pl.BlockSpec((T, H), lambda e: (0, 0)),
            pl.BlockSpec((1, H, F), lambda e: (e, 0, 0)),
            pl.BlockSpec((1, 1, F), lambda e: (e, 0, 0)),
            pl.BlockSpec((1, H, F), lambda e: (e, 0, 0)),
            pl.BlockSpec((1, 1, F), lambda e: (e, 0, 0)),
            pl.BlockSpec((1, F, H), lambda e: (e, 0, 0)),
            pl.BlockSpec((1, 1, H), lambda e: (e, 0, 0)),
        ],
        out_specs=pl.BlockSpec((T, H), lambda e: (0, 0)),
        out_shape=jax.ShapeDtypeStruct((T, H), jnp.float32),
        compiler_params=pltpu.CompilerParams(
            dimension_semantics=("arbitrary",),
        ),
    )(scores, hs, wg, bg, wu, bu, down_proj, bd)

    return out.reshape(B, S, H), scores


# speedup vs baseline: 2.5078x; 2.5078x over previous
import functools

import jax
import jax.numpy as jnp
from jax.experimental import pallas as pl
from jax.experimental.pallas import tpu as pltpu

HIDDEN = 768
INTER = 768
NUM_EXPERTS = 8
ALPHA = 1.702
LIMIT = 7.0
TOKENS = 2048
CHUNK = 512


def _router_body(x_ref, rw_ref, rb_ref, scores_ref):
    x = x_ref[...]
    rw = rw_ref[...]
    logits = jax.lax.dot_general(
        x, rw, (((1,), (1,)), ((), ())), preferred_element_type=jnp.float32
    ) + rb_ref[...]
    T, E = logits.shape
    eids = jax.lax.broadcasted_iota(jnp.int32, (T, E), 1)
    m1 = jnp.max(logits, axis=1, keepdims=True)
    i1 = jnp.min(jnp.where(logits == m1, eids, E), axis=1, keepdims=True)
    masked = jnp.where(eids == i1, -jnp.inf, logits)
    m2 = jnp.max(masked, axis=1, keepdims=True)
    i2 = jnp.min(jnp.where(masked == m2, eids, E), axis=1, keepdims=True)
    e2 = jnp.exp(m2 - m1)
    denom = 1.0 + e2
    p1 = 1.0 / denom
    p2 = e2 / denom
    scores_ref[...] = jnp.where(eids == i1, p1, 0.0) + jnp.where(eids == i2, p2, 0.0)


def _expert_body(scores_ref, x_ref, wg_ref, bg_ref, wu_ref, bu_ref,
                 wd_ref, bd_ref, out_ref):
    e = pl.program_id(0)
    wg = wg_ref[0]
    wu = wu_ref[0]
    wd = wd_ref[0]
    bg = bg_ref[0]
    bu = bu_ref[0]
    bd = bd_ref[0]
    T, E = scores_ref.shape
    eids = jax.lax.broadcasted_iota(jnp.int32, (T, E), 1)
    col_full = jnp.sum(
        jnp.where(eids == e, scores_ref[...], 0.0), axis=1, keepdims=True
    )
    for c in range(TOKENS // CHUNK):
        sl = pl.ds(c * CHUNK, CHUNK)
        xc = x_ref[sl, :]
        g = jax.lax.dot_general(
            xc, wg, (((1,), (0,)), ((), ())), preferred_element_type=jnp.float32
        ) + bg
        u = jax.lax.dot_general(
            xc, wu, (((1,), (0,)), ((), ())), preferred_element_type=jnp.float32
        ) + bu
        g = jnp.minimum(g, LIMIT)
        u = jnp.clip(u, -LIMIT, LIMIT)
        glu = g * jax.nn.sigmoid(g * ALPHA)
        h = (u + 1.0) * glu
        y = jax.lax.dot_general(
            h, wd, (((1,), (0,)), ((), ())), preferred_element_type=jnp.float32
        ) + bd
        y = y * col_full[c * CHUNK:(c + 1) * CHUNK, :]

        @pl.when(e == 0)
        def _():
            out_ref[sl, :] = y

        @pl.when(e > 0)
        def _():
            out_ref[sl, :] = out_ref[sl, :] + y


def kernel(hidden_states, router_weight, router_bias, gate_up_proj,
           gate_up_proj_bias, down_proj, down_proj_bias):
    B, S, H = hidden_states.shape
    T = B * S
    E = NUM_EXPERTS
    F = INTER
    hs = hidden_states.reshape(T, H)

    scores = pl.pallas_call(
        _router_body,
        out_shape=jax.ShapeDtypeStruct((T, E), jnp.float32),
    )(hs, router_weight, router_bias.reshape(1, E))

    wg = gate_up_proj[:, :, 0::2]
    wu = gate_up_proj[:, :, 1::2]
    bg = gate_up_proj_bias[:, 0::2].reshape(E, 1, F)
    bu = gate_up_proj_bias[:, 1::2].reshape(E, 1, F)
    bd = down_proj_bias.reshape(E, 1, H)

    out = pl.pallas_call(
        _expert_body,
        grid=(E,),
        in_specs=[
            pl.BlockSpec((T, E), lambda e: (0, 0)),
            pl.BlockSpec((T, H), lambda e: (0, 0)),
            pl.BlockSpec((1, H, F), lambda e: (e, 0, 0)),
            pl.BlockSpec((1, 1, F), lambda e: (e, 0, 0)),
            pl.BlockSpec((1, H, F), lambda e: (e, 0, 0)),
            pl.BlockSpec((1, 1, F), lambda e: (e, 0, 0)),
            pl.BlockSpec((1, F, H), lambda e: (e, 0, 0)),
            pl.BlockSpec((1, 1, H), lambda e: (e, 0, 0)),
        ],
        out_specs=pl.BlockSpec((T, H), lambda e: (0, 0)),
        out_shape=jax.ShapeDtypeStruct((T, H), jnp.float32),
        compiler_params=pltpu.CompilerParams(
            dimension_semantics=("arbitrary",),
        ),
    )(scores, hs, wg, bg, wu, bu, down_proj, bd)

    return out.reshape(B, S, H), scores


# speedup vs baseline: 4.3762x; 1.7451x over previous
import functools

import jax
import jax.numpy as jnp
from jax.experimental import pallas as pl
from jax.experimental.pallas import tpu as pltpu

HIDDEN = 768
INTER = 768
NUM_EXPERTS = 8
ALPHA = 1.702
LIMIT = 7.0
TOKENS = 2048
CHUNK = 512


def _router_body(x_ref, rw_ref, rb_ref, scores_ref):
    x = x_ref[...]
    rw = rw_ref[...]
    logits = jax.lax.dot_general(
        x, rw, (((1,), (1,)), ((), ())), preferred_element_type=jnp.float32
    ) + rb_ref[...]
    T, E = logits.shape
    eids = jax.lax.broadcasted_iota(jnp.int32, (T, E), 1)
    m1 = jnp.max(logits, axis=1, keepdims=True)
    i1 = jnp.min(jnp.where(logits == m1, eids, E), axis=1, keepdims=True)
    masked = jnp.where(eids == i1, -jnp.inf, logits)
    m2 = jnp.max(masked, axis=1, keepdims=True)
    i2 = jnp.min(jnp.where(masked == m2, eids, E), axis=1, keepdims=True)
    e2 = jnp.exp(m2 - m1)
    denom = 1.0 + e2
    p1 = 1.0 / denom
    p2 = e2 / denom
    scores_ref[...] = jnp.where(eids == i1, p1, 0.0) + jnp.where(eids == i2, p2, 0.0)


def _expert_body(scores_ref, x_ref, wg_ref, bg_ref, wu_ref, bu_ref,
                 wd_ref, bd_ref, out_ref):
    e = pl.program_id(0)
    wg = wg_ref[0]
    wu = wu_ref[0]
    wd = wd_ref[0]
    bg = bg_ref[0]
    bu = bu_ref[0]
    bd = bd_ref[0]
    T, E = scores_ref.shape
    eids = jax.lax.broadcasted_iota(jnp.int32, (T, E), 1)
    col_full = jnp.sum(
        jnp.where(eids == e, scores_ref[...], 0.0), axis=1, keepdims=True
    )
    for c in range(TOKENS // CHUNK):
        sl = pl.ds(c * CHUNK, CHUNK)
        xc = x_ref[sl, :]

        g = jax.lax.dot_general(
            xc, wg, (((1,), (0,)), ((), ())), preferred_element_type=jnp.float32
        ) + bg
        u = jax.lax.dot_general(
            xc, wu, (((1,), (0,)), ((), ())), preferred_element_type=jnp.float32
        ) + bu
        g = jnp.minimum(g, LIMIT)
        u = jnp.clip(u, -LIMIT, LIMIT)
        glu = g * jax.nn.sigmoid(g * ALPHA)
        h = ((u + 1.0) * glu).astype(jnp.bfloat16)
        y = jax.lax.dot_general(
            h, wd, (((1,), (0,)), ((), ())), preferred_element_type=jnp.float32
        ) + bd
        y = y * col_full[c * CHUNK:(c + 1) * CHUNK, :]

        @pl.when(e == 0)
        def _():
            out_ref[sl, :] = y

        @pl.when(e > 0)
        def _():
            out_ref[sl, :] = out_ref[sl, :] + y


def kernel(hidden_states, router_weight, router_bias, gate_up_proj,
           gate_up_proj_bias, down_proj, down_proj_bias):
    B, S, H = hidden_states.shape
    T = B * S
    E = NUM_EXPERTS
    F = INTER
    hs = hidden_states.reshape(T, H)

    scores = pl.pallas_call(
        _router_body,
        out_shape=jax.ShapeDtypeStruct((T, E), jnp.float32),
    )(hs, router_weight, router_bias.reshape(1, E))

    wg = gate_up_proj[:, :, 0::2].astype(jnp.bfloat16)
    wu = gate_up_proj[:, :, 1::2].astype(jnp.bfloat16)
    wd16 = down_proj.astype(jnp.bfloat16)
    hs16 = hs.astype(jnp.bfloat16)
    bg = gate_up_proj_bias[:, 0::2].reshape(E, 1, F)
    bu = gate_up_proj_bias[:, 1::2].reshape(E, 1, F)
    bd = down_proj_bias.reshape(E, 1, H)

    out = pl.pallas_call(
        _expert_body,
        grid=(E,),
        in_specs=[
            pl.BlockSpec((T, E), lambda e: (0, 0)),
            pl.BlockSpec((T, H), lambda e: (0, 0)),
            pl.BlockSpec((1, H, F), lambda e: (e, 0, 0)),
            pl.BlockSpec((1, 1, F), lambda e: (e, 0, 0)),
            pl.BlockSpec((1, H, F), lambda e: (e, 0, 0)),
            pl.BlockSpec((1, 1, F), lambda e: (e, 0, 0)),
            pl.BlockSpec((1, F, H), lambda e: (e, 0, 0)),
            pl.BlockSpec((1, 1, H), lambda e: (e, 0, 0)),
        ],
        out_specs=pl.BlockSpec((T, H), lambda e: (0, 0)),
        out_shape=jax.ShapeDtypeStruct((T, H), jnp.float32),
        compiler_params=pltpu.CompilerParams(
            dimension_semantics=("arbitrary",),
        ),
    )(scores, hs16, wg, bg, wu, bu, wd16, bd)

    return out.reshape(B, S, H), scores
